# baseline (speedup 1.0000x reference)
import functools

import jax
import jax.numpy as jnp
from jax import lax
from jax.experimental import pallas as pl
from jax.experimental.pallas import tpu as pltpu
from jax.experimental.pallas import tpu_sc as plsc

N = 10000
D = 128
K = 8
DD = 16
N_ITER = 3
N_LAYER = 2
TAU = 1.0

NC = 2
NS = 16
NW = NC * NS

EC = 56
PADR = 8
GE = 2
NBUF = 2
ROWS_PER_TILE = 624
RZ = 16


def _leaky(v):
    return jnp.where(v > 0, v, 0.01 * v)


def _norm_rows(v):
    outs = []
    for k in range(K):
        g = v[:, k * DD:(k + 1) * DD]
        ss = jnp.sum(g * g, axis=1, keepdims=True)
        nrm = jnp.sqrt(ss)
        outs.append(g / jnp.maximum(nrm, 1e-12))
    return jnp.concatenate(outs, axis=1)


_BR = 1000


def _tc_prep_body(x_ref, w_ref, b_ref, o_ref):
    h = jnp.dot(x_ref[...], w_ref[...], preferred_element_type=jnp.float32)
    h = _leaky(h + b_ref[...])
    o_ref[...] = _norm_rows(h)


def _tc_prep(x, W, b2):
    grid = N // _BR
    return pl.pallas_call(
        _tc_prep_body,
        grid=(grid,),
        in_specs=[
            pl.BlockSpec((_BR, D), lambda i: (i, 0)),
            pl.BlockSpec((D, D), lambda i: (0, 0)),
            pl.BlockSpec((1, D), lambda i: (0, 0)),
        ],
        out_specs=pl.BlockSpec((_BR, D), lambda i: (i, 0)),
        out_shape=jax.ShapeDtypeStruct((N, D), jnp.float32),
    )(x, W, b2)


def _tc_mid_body(s0_ref, s1_ref, xh_ref, o_ref):
    t = s0_ref[...] + s1_ref[...] + xh_ref[...]
    o_ref[...] = _norm_rows(t)


def _tc_mid(s0, s1, xh):
    grid = N // _BR
    spec = pl.BlockSpec((_BR, D), lambda i: (i, 0))
    return pl.pallas_call(
        _tc_mid_body,
        grid=(grid,),
        in_specs=[spec, spec, spec],
        out_specs=spec,
        out_shape=jax.ShapeDtypeStruct((N, D), jnp.float32),
    )(s0, s1, xh)


def _tc_end_body(s0_ref, s1_ref, xh_ref, h_ref, nx_ref):
    t = s0_ref[...] + s1_ref[...] + xh_ref[...]
    h = _leaky(_norm_rows(t))
    h_ref[...] = h
    nx_ref[...] = _norm_rows(h)


def _tc_end(s0, s1, xh):
    grid = N // _BR
    spec = pl.BlockSpec((_BR, D), lambda i: (i, 0))
    return pl.pallas_call(
        _tc_end_body,
        grid=(grid,),
        in_specs=[spec, spec, spec],
        out_specs=[spec, spec],
        out_shape=[
            jax.ShapeDtypeStruct((N, D), jnp.float32),
            jax.ShapeDtypeStruct((N, D), jnp.float32),
        ],
    )(s0, s1, xh)


NROT = 6


def _sc_body(nchunk, xh, uh, src3, trg3, out, acc, sidx, tidx,
             zb0, ub0, ob0, zb1, ub1, ob1,
             gsem0, gsem1, ssem0, ssem1, isem0, isem1):
    c = lax.axis_index("c")
    s = lax.axis_index("s")
    wid = c * NS + s

    zbs = (zb0, zb1)
    ubs = (ub0, ub1)
    obs = (ob0, ob1)
    gsems = (gsem0, gsem1)
    ssems = (ssem0, ssem1)
    isems = (isem0, isem1)

    def issue_idx(i, p):
        pltpu.async_copy(src3.at[wid, i], sidx.at[lax.rem(i, NROT)], isems[p])
        pltpu.async_copy(trg3.at[wid, i], tidx.at[lax.rem(i, NROT)], isems[p])

    def wait_idx(i, p):
        r = lax.rem(i, NROT)
        pltpu.make_async_copy(src3.at[wid, i], sidx.at[r], isems[p]).wait()
        pltpu.make_async_copy(trg3.at[wid, i], tidx.at[r], isems[p]).wait()

    def issue_gathers(i, b):
        r = lax.rem(i, NROT)
        pltpu.async_copy(xh.at[sidx.at[r]], zbs[b], gsems[b])
        pltpu.async_copy(uh.at[tidx.at[r]], ubs[b], gsems[b])

    def wait_gathers(i, b):
        r = lax.rem(i, NROT)
        pltpu.make_async_copy(xh.at[sidx.at[r]], zbs[b], gsems[b]).wait()
        pltpu.make_async_copy(uh.at[tidx.at[r]], ubs[b], gsems[b]).wait()

    issue_idx(0, 0)
    issue_idx(1, 1)
    wait_idx(0, 0)
    issue_gathers(0, 0)

    zero = jnp.zeros((DD,), jnp.float32)
    for r in range(RZ):
        for k in range(K):
            ob0[r, pl.ds(k * DD, DD)] = zero

    row0 = s * ROWS_PER_TILE

    def zero_body(i, _):
        pltpu.sync_copy(ob0.at[pl.ds(0, RZ)], acc.at[pl.ds(row0 + i * RZ, RZ)])
        return 0

    lax.fori_loop(0, ROWS_PER_TILE // RZ, zero_body, 0)

    @pl.when(s == NS - 1)
    def _():
        pltpu.sync_copy(ob0.at[pl.ds(0, RZ)],
                        acc.at[pl.ds(NS * ROWS_PER_TILE, RZ)])

    plsc.subcore_barrier()

    lanes = lax.iota(jnp.int32, DD)

    def _permx(v, sh):
        return v.at[jnp.bitwise_xor(lanes, sh)].get(mode="promise_in_bounds")

    mask_half = lanes < 8
    mask_q = (lanes & 4) == 0
    mask_p = (lanes & 2) == 0
    ext = [jnp.full((DD,), 2 * (((k & 1) << 2) | (k & 2) | ((k & 4) >> 2)),
                    jnp.int32) for k in range(K)]

    def _comb(a, b2, mask, sh):
        s1 = jnp.where(mask, a, b2)
        s2 = jnp.where(mask, b2, a)
        return s1 + _permx(s2, sh)

    def _pack3(q1):
        w = [_comb(q1[2 * j2], q1[2 * j2 + 1], mask_half, 8)
             for j2 in range(4)]
        y = [_comb(w[0], w[1], mask_q, 4), _comb(w[2], w[3], mask_q, 4)]
        return _comb(y[0], y[1], mask_p, 2)

    def _make_group_body(zb, ub, ob):
        def group_body(g, _):
            for j in range(GE):
                row = g * GE + j
                zr = [zb[row, pl.ds(k * DD, DD)] for k in range(K)]
                ur = [ub[row, pl.ds(k * DD, DD)] for k in range(K)]
                f = _pack3([zr[k] * ur[k] for k in range(K)])
                p = (f + _permx(f, 1)) * (1.0 / TAU)
                m = p
                for sh in (8, 4, 2):
                    m = jnp.maximum(m, _permx(m, sh))
                e = jnp.exp(p - m)
                tot = e
                for sh in (8, 4, 2):
                    tot = tot + _permx(tot, sh)
                r = e * (1.0 / tot)
                for k in range(K):
                    rk = r.at[ext[k]].get(mode="promise_in_bounds")
                    ob[row, pl.ds(k * DD, DD)] = zr[k] * rk
            return 0
        return group_body

    group_bodies = [_make_group_body(zbs[b], ubs[b], obs[b])
                    for b in range(NBUF)]

    def pair_body(g, _):
        for b in range(NBUF):
            i = g * NBUF + b

            @pl.when(i + 1 < nchunk)
            def _():
                wait_idx(i + 1, 1 - b)
                issue_gathers(i + 1, 1 - b)

            wait_gathers(i, b)

            @pl.when(i >= NBUF)
            def _():
                pltpu.make_async_copy(
                    obs[b], acc.at[tidx.at[lax.rem(i - NBUF, NROT)]],
                    ssems[b]).wait()

            lax.fori_loop(0, EC // GE, group_bodies[b], 0)

            pltpu.async_copy(obs[b], acc.at[tidx.at[lax.rem(i, NROT)]],
                             ssems[b], add=True)

            @pl.when(i + 2 < nchunk)
            def _():
                issue_idx(i + 2, b)
        return 0

    lax.fori_loop(0, nchunk // NBUF, pair_body, 0)

    for b in range(NBUF):
        i = nchunk - NBUF + b
        pltpu.make_async_copy(
            obs[b], acc.at[tidx.at[lax.rem(i, NROT)]], ssems[b]).wait()

    plsc.subcore_barrier()
    pltpu.sync_copy(acc.at[pl.ds(row0, ROWS_PER_TILE)],
                    out.at[c, pl.ds(row0, ROWS_PER_TILE)])

    @pl.when(s == NS - 1)
    def _():
        pltpu.sync_copy(acc.at[pl.ds(NS * ROWS_PER_TILE, RZ)],
                        out.at[c, pl.ds(NS * ROWS_PER_TILE, RZ)])


def _sc_routing(xh, uh, src3, trg3):
    nchunk = src3.shape[1]
    mesh = plsc.VectorSubcoreMesh(core_axis_name="c", subcore_axis_name="s")
    kern = pl.kernel(
        functools.partial(_sc_body, nchunk),
        out_type=jax.ShapeDtypeStruct((NC, N, D), jnp.float32),
        mesh=mesh,
        scratch_types=[
            pltpu.VMEM_SHARED((N + PADR, D), jnp.float32),
            pltpu.VMEM((NROT, EC), jnp.int32),
            pltpu.VMEM((NROT, EC), jnp.int32),
            pltpu.VMEM((EC, D), jnp.float32),
            pltpu.VMEM((EC, D), jnp.float32),
            pltpu.VMEM((EC, D), jnp.float32),
            pltpu.VMEM((EC, D), jnp.float32),
            pltpu.VMEM((EC, D), jnp.float32),
            pltpu.VMEM((EC, D), jnp.float32),
            pltpu.SemaphoreType.DMA,
            pltpu.SemaphoreType.DMA,
            pltpu.SemaphoreType.DMA,
            pltpu.SemaphoreType.DMA,
            pltpu.SemaphoreType.DMA,
            pltpu.SemaphoreType.DMA,
        ],
    )
    return kern(xh, uh, src3, trg3)


def _padrows(v):
    return jnp.pad(v, ((0, PADR), (0, 0)))


def kernel(x, src_trg, W, b):
    m = src_trg.shape[1]
    nchunk = -(-m // (NW * EC))
    nchunk += nchunk % 2
    mp = NW * EC * nchunk
    src = jnp.concatenate(
        [src_trg[0].astype(jnp.int32), jnp.zeros((mp - m,), jnp.int32)])
    trg = jnp.concatenate(
        [src_trg[1].astype(jnp.int32), jnp.full((mp - m,), N, jnp.int32)])
    src = src.reshape(NW, nchunk, EC)
    trg = trg.reshape(NW, nchunk, EC)
    b2 = b.reshape(1, D)
    xh = _tc_prep(x.astype(jnp.float32), W, b2)
    xhp = _padrows(xh)
    h_out = None
    for _ in range(N_LAYER):
        up = xhp
        for it in range(N_ITER):
            S = _sc_routing(xhp, up, src, trg)
            if it < N_ITER - 1:
                up = _padrows(_tc_mid(S[0], S[1], xh))
            else:
                h_out, xh = _tc_end(S[0], S[1], xh)
                xhp = _padrows(xh)
    return h_out


# speedup vs baseline: 1.1260x; 1.1260x over previous
import functools

import jax
import jax.numpy as jnp
from jax import lax
from jax.experimental import pallas as pl
from jax.experimental.pallas import tpu as pltpu
from jax.experimental.pallas import tpu_sc as plsc

N = 10000
D = 128
K = 8
DD = 16
N_ITER = 3
N_LAYER = 2
TAU = 1.0

NC = 2
NS = 16
NW = NC * NS

EC = 40
PADR = 8
GE = 2
NBUF = 2
ROWS_PER_TILE = 624
RZ = 16


def _leaky(v):
    return jnp.where(v > 0, v, 0.01 * v)


def _norm_rows(v):
    outs = []
    for k in range(K):
        g = v[:, k * DD:(k + 1) * DD]
        ss = jnp.sum(g * g, axis=1, keepdims=True)
        nrm = jnp.sqrt(ss)
        outs.append(g / jnp.maximum(nrm, 1e-12))
    return jnp.concatenate(outs, axis=1)


_BR = 1000


def _tc_prep_body(x_ref, w_ref, b_ref, o_ref):
    h = jnp.dot(x_ref[...], w_ref[...], preferred_element_type=jnp.float32)
    h = _leaky(h + b_ref[...])
    o_ref[...] = _norm_rows(h)


def _tc_prep(x, W, b2):
    grid = N // _BR
    return pl.pallas_call(
        _tc_prep_body,
        grid=(grid,),
        in_specs=[
            pl.BlockSpec((_BR, D), lambda i: (i, 0)),
            pl.BlockSpec((D, D), lambda i: (0, 0)),
            pl.BlockSpec((1, D), lambda i: (0, 0)),
        ],
        out_specs=pl.BlockSpec((_BR, D), lambda i: (i, 0)),
        out_shape=jax.ShapeDtypeStruct((N, D), jnp.float32),
    )(x, W, b2)


def _tc_mid_body(s0_ref, s1_ref, xh_ref, o_ref):
    t = s0_ref[...] + s1_ref[...] + xh_ref[...]
    o_ref[...] = _norm_rows(t)


def _tc_mid(s0, s1, xh):
    grid = N // _BR
    spec = pl.BlockSpec((_BR, D), lambda i: (i, 0))
    return pl.pallas_call(
        _tc_mid_body,
        grid=(grid,),
        in_specs=[spec, spec, spec],
        out_specs=spec,
        out_shape=jax.ShapeDtypeStruct((N, D), jnp.float32),
    )(s0, s1, xh)


def _tc_end_body(s0_ref, s1_ref, xh_ref, h_ref, nx_ref):
    t = s0_ref[...] + s1_ref[...] + xh_ref[...]
    h = _leaky(_norm_rows(t))
    h_ref[...] = h
    nx_ref[...] = _norm_rows(h)


def _tc_end(s0, s1, xh):
    grid = N // _BR
    spec = pl.BlockSpec((_BR, D), lambda i: (i, 0))
    return pl.pallas_call(
        _tc_end_body,
        grid=(grid,),
        in_specs=[spec, spec, spec],
        out_specs=[spec, spec],
        out_shape=[
            jax.ShapeDtypeStruct((N, D), jnp.float32),
            jax.ShapeDtypeStruct((N, D), jnp.float32),
        ],
    )(s0, s1, xh)


NROT = 6


def _sc_body(nchunk, xh, uh, src3, trg3, out, acc, sidx, tidx,
             zb0, ub0, ob0, zb1, ub1, ob1,
             gsem0, gsem1, ssem0, ssem1, isem0, isem1):
    c = lax.axis_index("c")
    s = lax.axis_index("s")
    wid = c * NS + s

    zbs = (zb0, zb1)
    ubs = (ub0, ub1)
    obs = (ob0, ob1)
    gsems = (gsem0, gsem1)
    ssems = (ssem0, ssem1)
    isems = (isem0, isem1)

    def issue_idx(i, p):
        pltpu.async_copy(src3.at[wid, i], sidx.at[lax.rem(i, NROT)], isems[p])
        pltpu.async_copy(trg3.at[wid, i], tidx.at[lax.rem(i, NROT)], isems[p])

    def wait_idx(i, p):
        r = lax.rem(i, NROT)
        pltpu.make_async_copy(src3.at[wid, i], sidx.at[r], isems[p]).wait()
        pltpu.make_async_copy(trg3.at[wid, i], tidx.at[r], isems[p]).wait()

    def issue_gathers(i, b):
        r = lax.rem(i, NROT)
        pltpu.async_copy(xh.at[sidx.at[r]], zbs[b], gsems[b])
        pltpu.async_copy(uh.at[tidx.at[r]], ubs[b], gsems[b])

    def wait_gathers(i, b):
        r = lax.rem(i, NROT)
        pltpu.make_async_copy(xh.at[sidx.at[r]], zbs[b], gsems[b]).wait()
        pltpu.make_async_copy(uh.at[tidx.at[r]], ubs[b], gsems[b]).wait()

    issue_idx(0, 0)
    issue_idx(1, 1)
    wait_idx(0, 0)
    issue_gathers(0, 0)

    zero = jnp.zeros((DD,), jnp.float32)
    for r in range(RZ):
        for k in range(K):
            ob0[r, pl.ds(k * DD, DD)] = zero

    row0 = s * ROWS_PER_TILE

    def zero_body(i, _):
        pltpu.sync_copy(ob0.at[pl.ds(0, RZ)], acc.at[pl.ds(row0 + i * RZ, RZ)])
        return 0

    lax.fori_loop(0, ROWS_PER_TILE // RZ, zero_body, 0)

    @pl.when(s == NS - 1)
    def _():
        pltpu.sync_copy(ob0.at[pl.ds(0, RZ)],
                        acc.at[pl.ds(NS * ROWS_PER_TILE, RZ)])

    plsc.subcore_barrier()

    lanes = lax.iota(jnp.int32, DD)

    def _permx(v, sh):
        return v.at[jnp.bitwise_xor(lanes, sh)].get(mode="promise_in_bounds")

    mask_half = lanes < 8
    mask_q = (lanes & 4) == 0
    mask_p = (lanes & 2) == 0
    ext = [jnp.full((DD,), 2 * (((k & 1) << 2) | (k & 2) | ((k & 4) >> 2)),
                    jnp.int32) for k in range(K)]

    def _comb(a, b2, mask, sh):
        s1 = jnp.where(mask, a, b2)
        s2 = jnp.where(mask, b2, a)
        return s1 + _permx(s2, sh)

    def _pack3(q1):
        w = [_comb(q1[2 * j2], q1[2 * j2 + 1], mask_half, 8)
             for j2 in range(4)]
        y = [_comb(w[0], w[1], mask_q, 4), _comb(w[2], w[3], mask_q, 4)]
        return _comb(y[0], y[1], mask_p, 2)

    def _make_group_body(zb, ub, ob):
        def group_body(g, _):
            for j in range(GE):
                row = g * GE + j
                zr = [zb[row, pl.ds(k * DD, DD)] for k in range(K)]
                ur = [ub[row, pl.ds(k * DD, DD)] for k in range(K)]
                f = _pack3([zr[k] * ur[k] for k in range(K)])
                p = (f + _permx(f, 1)) * (1.0 / TAU)
                m = p
                for sh in (8, 4, 2):
                    m = jnp.maximum(m, _permx(m, sh))
                e = jnp.exp(p - m)
                tot = e
                for sh in (8, 4, 2):
                    tot = tot + _permx(tot, sh)
                r = e * (1.0 / tot)
                for k in range(K):
                    rk = r.at[ext[k]].get(mode="promise_in_bounds")
                    ob[row, pl.ds(k * DD, DD)] = zr[k] * rk
            return 0
        return group_body

    group_bodies = [_make_group_body(zbs[b], ubs[b], obs[b])
                    for b in range(NBUF)]

    def pair_body(g, _):
        for b in range(NBUF):
            i = g * NBUF + b

            @pl.when(i + 1 < nchunk)
            def _():
                wait_idx(i + 1, 1 - b)
                issue_gathers(i + 1, 1 - b)

            wait_gathers(i, b)

            @pl.when(i >= NBUF)
            def _():
                pltpu.make_async_copy(
                    obs[b], acc.at[tidx.at[lax.rem(i - NBUF, NROT)]],
                    ssems[b]).wait()

            lax.fori_loop(0, EC // GE, group_bodies[b], 0)

            pltpu.async_copy(obs[b], acc.at[tidx.at[lax.rem(i, NROT)]],
                             ssems[b], add=True)

            @pl.when(i + 2 < nchunk)
            def _():
                issue_idx(i + 2, b)
        return 0

    lax.fori_loop(0, nchunk // NBUF, pair_body, 0)

    for b in range(NBUF):
        i = nchunk - NBUF + b
        pltpu.make_async_copy(
            obs[b], acc.at[tidx.at[lax.rem(i, NROT)]], ssems[b]).wait()

    plsc.subcore_barrier()
    pltpu.sync_copy(acc.at[pl.ds(row0, ROWS_PER_TILE)],
                    out.at[c, pl.ds(row0, ROWS_PER_TILE)])

    @pl.when(s == NS - 1)
    def _():
        pltpu.sync_copy(acc.at[pl.ds(NS * ROWS_PER_TILE, RZ)],
                        out.at[c, pl.ds(NS * ROWS_PER_TILE, RZ)])


def _sc_routing(xh, uh, src3, trg3):
    nchunk = src3.shape[1]
    mesh = plsc.VectorSubcoreMesh(core_axis_name="c", subcore_axis_name="s")
    kern = pl.kernel(
        functools.partial(_sc_body, nchunk),
        out_type=jax.ShapeDtypeStruct((NC, N, D), jnp.float32),
        mesh=mesh,
        scratch_types=[
            pltpu.VMEM_SHARED((N + PADR, D), jnp.float32),
            pltpu.VMEM((NROT, EC), jnp.int32),
            pltpu.VMEM((NROT, EC), jnp.int32),
            pltpu.VMEM((EC, D), jnp.float32),
            pltpu.VMEM((EC, D), jnp.float32),
            pltpu.VMEM((EC, D), jnp.float32),
            pltpu.VMEM((EC, D), jnp.float32),
            pltpu.VMEM((EC, D), jnp.float32),
            pltpu.VMEM((EC, D), jnp.float32),
            pltpu.SemaphoreType.DMA,
            pltpu.SemaphoreType.DMA,
            pltpu.SemaphoreType.DMA,
            pltpu.SemaphoreType.DMA,
            pltpu.SemaphoreType.DMA,
            pltpu.SemaphoreType.DMA,
        ],
    )
    return kern(xh, uh, src3, trg3)


def _padrows(v):
    return jnp.pad(v, ((0, PADR), (0, 0)))


def kernel(x, src_trg, W, b):
    m = src_trg.shape[1]
    nchunk = -(-m // (NW * EC))
    nchunk += nchunk % 2
    mp = NW * EC * nchunk
    if mp > m:
        src = jnp.concatenate(
            [src_trg[0].astype(jnp.int32), jnp.zeros((mp - m,), jnp.int32)])
        trg = jnp.concatenate(
            [src_trg[1].astype(jnp.int32), jnp.full((mp - m,), N, jnp.int32)])
        pad_tables = _padrows
    else:
        src = src_trg[0].astype(jnp.int32)
        trg = src_trg[1].astype(jnp.int32)
        pad_tables = lambda v: v
    src = src.reshape(NW, nchunk, EC)
    trg = trg.reshape(NW, nchunk, EC)
    b2 = b.reshape(1, D)
    xh = _tc_prep(x.astype(jnp.float32), W, b2)
    xhp = pad_tables(xh)
    h_out = None
    for _ in range(N_LAYER):
        up = xhp
        for it in range(N_ITER):
            S = _sc_routing(xhp, up, src, trg)
            if it < N_ITER - 1:
                up = pad_tables(_tc_mid(S[0], S[1], xh))
            else:
                h_out, xh = _tc_end(S[0], S[1], xh)
                xhp = pad_tables(xh)
    return h_out


# speedup vs baseline: 1.1770x; 1.0452x over previous
import functools

import jax
import jax.numpy as jnp
from jax import lax
from jax.experimental import pallas as pl
from jax.experimental.pallas import tpu as pltpu
from jax.experimental.pallas import tpu_sc as plsc

N = 10000
D = 128
K = 8
DD = 16
N_ITER = 3
N_LAYER = 2
TAU = 1.0

NC = 2
NS = 16
NW = NC * NS

EC = 40
PADR = 8
GE = 2
NBUF = 2
ROWS_PER_TILE = 624
RZ = 16


def _leaky(v):
    return jnp.where(v > 0, v, 0.01 * v)


def _norm_rows(v):
    outs = []
    for k in range(K):
        g = v[:, k * DD:(k + 1) * DD]
        ss = jnp.sum(g * g, axis=1, keepdims=True)
        nrm = jnp.sqrt(ss)
        outs.append(g / jnp.maximum(nrm, 1e-12))
    return jnp.concatenate(outs, axis=1)


_BR = 1000


def _tc_prep_body(x_ref, w_ref, b_ref, o_ref):
    h = jnp.dot(x_ref[...], w_ref[...], preferred_element_type=jnp.float32)
    h = _leaky(h + b_ref[...])
    o_ref[...] = _norm_rows(h)


def _tc_prep(x, W, b2):
    grid = N // _BR
    return pl.pallas_call(
        _tc_prep_body,
        grid=(grid,),
        in_specs=[
            pl.BlockSpec((_BR, D), lambda i: (i, 0)),
            pl.BlockSpec((D, D), lambda i: (0, 0)),
            pl.BlockSpec((1, D), lambda i: (0, 0)),
        ],
        out_specs=pl.BlockSpec((_BR, D), lambda i: (i, 0)),
        out_shape=jax.ShapeDtypeStruct((N, D), jnp.float32),
    )(x, W, b2)


def _tc_mid_body(s0_ref, s1_ref, xh_ref, o_ref):
    t = s0_ref[...] + s1_ref[...] + xh_ref[...]
    o_ref[...] = _norm_rows(t)


def _tc_mid(s0, s1, xh):
    grid = N // _BR
    spec = pl.BlockSpec((_BR, D), lambda i: (i, 0))
    return pl.pallas_call(
        _tc_mid_body,
        grid=(grid,),
        in_specs=[spec, spec, spec],
        out_specs=spec,
        out_shape=jax.ShapeDtypeStruct((N, D), jnp.float32),
    )(s0, s1, xh)


def _tc_end_body(s0_ref, s1_ref, xh_ref, h_ref, nx_ref):
    t = s0_ref[...] + s1_ref[...] + xh_ref[...]
    h = _leaky(_norm_rows(t))
    h_ref[...] = h
    nx_ref[...] = _norm_rows(h)


def _tc_end(s0, s1, xh):
    grid = N // _BR
    spec = pl.BlockSpec((_BR, D), lambda i: (i, 0))
    return pl.pallas_call(
        _tc_end_body,
        grid=(grid,),
        in_specs=[spec, spec, spec],
        out_specs=[spec, spec],
        out_shape=[
            jax.ShapeDtypeStruct((N, D), jnp.float32),
            jax.ShapeDtypeStruct((N, D), jnp.float32),
        ],
    )(s0, s1, xh)


NROT = 6


def _sc_body(nchunk, xh, uh, src3, trg3, out, acc, sidx, tidx,
             zb0, ub0, ob0, zb1, ub1, ob1,
             gsem0, gsem1, ssem0, ssem1, isem0, isem1):
    c = lax.axis_index("c")
    s = lax.axis_index("s")
    wid = c * NS + s

    zbs = (zb0, zb1)
    ubs = (ub0, ub1)
    obs = (ob0, ob1)
    gsems = (gsem0, gsem1)
    ssems = (ssem0, ssem1)
    isems = (isem0, isem1)

    def issue_idx(i, p):
        pltpu.async_copy(src3.at[wid, i], sidx.at[lax.rem(i, NROT)], isems[p])
        pltpu.async_copy(trg3.at[wid, i], tidx.at[lax.rem(i, NROT)], isems[p])

    def wait_idx(i, p):
        r = lax.rem(i, NROT)
        pltpu.make_async_copy(src3.at[wid, i], sidx.at[r], isems[p]).wait()
        pltpu.make_async_copy(trg3.at[wid, i], tidx.at[r], isems[p]).wait()

    def issue_gathers(i, b):
        r = lax.rem(i, NROT)
        pltpu.async_copy(xh.at[sidx.at[r]], zbs[b], gsems[b])
        pltpu.async_copy(uh.at[tidx.at[r]], ubs[b], gsems[b])

    def wait_gathers(i, b):
        r = lax.rem(i, NROT)
        pltpu.make_async_copy(xh.at[sidx.at[r]], zbs[b], gsems[b]).wait()
        pltpu.make_async_copy(uh.at[tidx.at[r]], ubs[b], gsems[b]).wait()

    issue_idx(0, 0)
    issue_idx(1, 1)
    wait_idx(0, 0)
    issue_gathers(0, 0)

    zero = jnp.zeros((DD,), jnp.float32)
    for r in range(RZ):
        for k in range(K):
            ob0[r, pl.ds(k * DD, DD)] = zero

    row0 = s * ROWS_PER_TILE

    def zero_body(i, _):
        pltpu.sync_copy(ob0.at[pl.ds(0, RZ)], acc.at[pl.ds(row0 + i * RZ, RZ)])
        return 0

    lax.fori_loop(0, ROWS_PER_TILE // RZ, zero_body, 0)

    @pl.when(s == NS - 1)
    def _():
        pltpu.sync_copy(ob0.at[pl.ds(0, RZ)],
                        acc.at[pl.ds(NS * ROWS_PER_TILE, RZ)])

    plsc.subcore_barrier()

    lanes = lax.iota(jnp.int32, DD)

    def _permx(v, sh):
        return v.at[jnp.bitwise_xor(lanes, sh)].get(mode="promise_in_bounds")

    mask_half = lanes < 8
    mask_q = (lanes & 4) == 0
    mask_p = (lanes & 2) == 0
    ext = [jnp.full((DD,), 2 * (((k & 1) << 2) | (k & 2) | ((k & 4) >> 2)),
                    jnp.int32) for k in range(K)]

    def _comb(a, b2, mask, sh):
        s1 = jnp.where(mask, a, b2)
        s2 = jnp.where(mask, b2, a)
        return s1 + _permx(s2, sh)

    def _pack3(q1):
        w = [_comb(q1[2 * j2], q1[2 * j2 + 1], mask_half, 8)
             for j2 in range(4)]
        y = [_comb(w[0], w[1], mask_q, 4), _comb(w[2], w[3], mask_q, 4)]
        return _comb(y[0], y[1], mask_p, 2)

    def _make_group_body(zb, ub, ob):
        def group_body(g, _):
            for j in range(GE):
                row = g * GE + j
                zr = [zb[row, pl.ds(k * DD, DD)] for k in range(K)]
                ur = [ub[row, pl.ds(k * DD, DD)] for k in range(K)]
                f = _pack3([zr[k] * ur[k] for k in range(K)])
                p = (f + _permx(f, 1)) * (1.0 / TAU)
                m = p
                for sh in (8, 4, 2):
                    m = jnp.maximum(m, _permx(m, sh))
                e = jnp.exp(p - m)
                tot = e
                for sh in (8, 4, 2):
                    tot = tot + _permx(tot, sh)
                r = e * (1.0 / tot)
                for k in range(K):
                    rk = r.at[ext[k]].get(mode="promise_in_bounds")
                    z2 = zb[row, pl.ds(k * DD, DD)]
                    ob[row, pl.ds(k * DD, DD)] = z2 * rk
            return 0
        return group_body

    group_bodies = [_make_group_body(zbs[b], ubs[b], obs[b])
                    for b in range(NBUF)]

    def pair_body(g, _):
        for b in range(NBUF):
            i = g * NBUF + b

            @pl.when(i + 1 < nchunk)
            def _():
                wait_idx(i + 1, 1 - b)
                issue_gathers(i + 1, 1 - b)

            wait_gathers(i, b)

            @pl.when(i >= NBUF)
            def _():
                pltpu.make_async_copy(
                    obs[b], acc.at[tidx.at[lax.rem(i - NBUF, NROT)]],
                    ssems[b]).wait()

            lax.fori_loop(0, EC // GE, group_bodies[b], 0)

            pltpu.async_copy(obs[b], acc.at[tidx.at[lax.rem(i, NROT)]],
                             ssems[b], add=True)

            @pl.when(i + 2 < nchunk)
            def _():
                issue_idx(i + 2, b)
        return 0

    lax.fori_loop(0, nchunk // NBUF, pair_body, 0)

    for b in range(NBUF):
        i = nchunk - NBUF + b
        pltpu.make_async_copy(
            obs[b], acc.at[tidx.at[lax.rem(i, NROT)]], ssems[b]).wait()

    plsc.subcore_barrier()
    pltpu.sync_copy(acc.at[pl.ds(row0, ROWS_PER_TILE)],
                    out.at[c, pl.ds(row0, ROWS_PER_TILE)])

    @pl.when(s == NS - 1)
    def _():
        pltpu.sync_copy(acc.at[pl.ds(NS * ROWS_PER_TILE, RZ)],
                        out.at[c, pl.ds(NS * ROWS_PER_TILE, RZ)])


def _sc_routing(xh, uh, src3, trg3):
    nchunk = src3.shape[1]
    mesh = plsc.VectorSubcoreMesh(core_axis_name="c", subcore_axis_name="s")
    kern = pl.kernel(
        functools.partial(_sc_body, nchunk),
        out_type=jax.ShapeDtypeStruct((NC, N, D), jnp.float32),
        mesh=mesh,
        scratch_types=[
            pltpu.VMEM_SHARED((N + PADR, D), jnp.float32),
            pltpu.VMEM((NROT, EC), jnp.int32),
            pltpu.VMEM((NROT, EC), jnp.int32),
            pltpu.VMEM((EC, D), jnp.float32),
            pltpu.VMEM((EC, D), jnp.float32),
            pltpu.VMEM((EC, D), jnp.float32),
            pltpu.VMEM((EC, D), jnp.float32),
            pltpu.VMEM((EC, D), jnp.float32),
            pltpu.VMEM((EC, D), jnp.float32),
            pltpu.SemaphoreType.DMA,
            pltpu.SemaphoreType.DMA,
            pltpu.SemaphoreType.DMA,
            pltpu.SemaphoreType.DMA,
            pltpu.SemaphoreType.DMA,
            pltpu.SemaphoreType.DMA,
        ],
    )
    return kern(xh, uh, src3, trg3)


def _padrows(v):
    return jnp.pad(v, ((0, PADR), (0, 0)))


def kernel(x, src_trg, W, b):
    m = src_trg.shape[1]
    nchunk = -(-m // (NW * EC))
    nchunk += nchunk % 2
    mp = NW * EC * nchunk
    if mp > m:
        src = jnp.concatenate(
            [src_trg[0].astype(jnp.int32), jnp.zeros((mp - m,), jnp.int32)])
        trg = jnp.concatenate(
            [src_trg[1].astype(jnp.int32), jnp.full((mp - m,), N, jnp.int32)])
        pad_tables = _padrows
    else:
        src = src_trg[0].astype(jnp.int32)
        trg = src_trg[1].astype(jnp.int32)
        pad_tables = lambda v: v
    src = src.reshape(NW, nchunk, EC)
    trg = trg.reshape(NW, nchunk, EC)
    b2 = b.reshape(1, D)
    xh = _tc_prep(x.astype(jnp.float32), W, b2)
    xhp = pad_tables(xh)
    h_out = None
    for _ in range(N_LAYER):
        up = xhp
        for it in range(N_ITER):
            S = _sc_routing(xhp, up, src, trg)
            if it < N_ITER - 1:
                up = pad_tables(_tc_mid(S[0], S[1], xh))
            else:
                h_out, xh = _tc_end(S[0], S[1], xh)
                xhp = pad_tables(xh)
    return h_out
